# baseline (speedup 1.0000x reference)
import functools

import jax
import jax.numpy as jnp
from jax import lax
from jax.experimental import pallas as pl
from jax.experimental.pallas import tpu as pltpu
from jax.experimental.pallas import tpu_sc as plsc

D = 64
EPS = 1e-5
L = 16
NC = 2
NS = 16
NW = NC * NS
NB = 4
G1 = 128


def _rsqrt_vec(x):
    i = lax.bitcast_convert_type(x, jnp.int32)
    i = jnp.int32(0x5F375A86) - lax.shift_right_logical(i, 1)
    y = lax.bitcast_convert_type(i, jnp.float32)
    for _ in range(3):
        y = y * (jnp.float32(1.5) - jnp.float32(0.5) * x * y * y)
    return y


def _make_kernel(batch, hist):
    bpw = batch // NW
    g2 = hist - G1
    hpad = ((hist + L - 1) // L) * L
    mesh = plsc.VectorSubcoreMesh(
        core_axis_name="c", subcore_axis_name="s", num_cores=NC, num_subcores=NS
    )

    @functools.partial(
        pl.kernel,
        out_type=jax.ShapeDtypeStruct((batch, hist, D), jnp.float32),
        mesh=mesh,
        scratch_types=[
            pltpu.VMEM((bpw, hist), jnp.int32),
            pltpu.VMEM((NB, hpad, D), jnp.float32),
            pltpu.VMEM((2, D), jnp.float32),
        ]
        + [pltpu.SemaphoreType.DMA] * (2 * NB),
        compiler_params=pltpu.CompilerParams(
            use_tc_tiling_on_sc=False, needs_layout_passes=False
        ),
    )
    def k(x_hbm, table_hbm, gb_hbm, out_hbm, idx_all, rows, gb_v, *sems):
        gsem = sems[:NB]
        osem = sems[NB:]
        wid = lax.axis_index("s") * NC + lax.axis_index("c")
        base = wid * bpw
        pltpu.sync_copy(x_hbm.at[pl.ds(base, bpw)], idx_all)
        pltpu.sync_copy(gb_hbm, gb_v)

        lane = lax.iota(jnp.int32, L)
        gvecs = [gb_v[0, pl.ds(i * L, L)] for i in range(D // L)]
        bvecs = [gb_v[1, pl.ds(i * L, L)] for i in range(D // L)]

        def start_gather(b, j):
            pltpu.async_copy(
                table_hbm.at[idx_all.at[j, pl.ds(0, G1)]],
                rows.at[b].at[pl.ds(0, G1)],
                gsem[b],
            )
            pltpu.async_copy(
                table_hbm.at[idx_all.at[j, pl.ds(G1, g2)]],
                rows.at[b].at[pl.ds(G1, g2)],
                gsem[b],
            )

        def wait_gather(b):
            pltpu.make_async_copy(
                table_hbm.at[pl.ds(0, hist)],
                rows.at[b].at[pl.ds(0, hist)],
                gsem[b],
            ).wait()

        def start_out(b, j):
            pltpu.async_copy(
                rows.at[b].at[pl.ds(0, hist)], out_hbm.at[base + j], osem[b]
            )

        def wait_out(b):
            pltpu.make_async_copy(
                rows.at[b].at[pl.ds(0, hist)], out_hbm.at[0], osem[b]
            ).wait()

        def compute(rows_ref):
            def grp_body(g, carry):
                row_ids = g * L + lane
                fzero = lax.convert_element_type(lane, jnp.float32) * 0.0
                acc = [fzero] * 4
                acc2 = [fzero] * 4
                for c in range(D):
                    col = lax.bitwise_and(lane + c, jnp.int32(D - 1))
                    v = plsc.load_gather(rows_ref, [row_ids, col])
                    r = c % 4
                    acc[r] = acc[r] + v
                    acc2[r] = acc2[r] + v * v
                s = (acc[0] + acc[1]) + (acc[2] + acc[3])
                q = (acc2[0] + acc2[1]) + (acc2[2] + acc2[3])
                mean = s * jnp.float32(1.0 / D)
                var = q * jnp.float32(1.0 / D) - mean * mean
                kk = _rsqrt_vec(var + jnp.float32(EPS))
                off = mean * kk
                for c in range(D):
                    col = lax.bitwise_and(lane + c, jnp.int32(D - 1))
                    v = plsc.load_gather(rows_ref, [row_ids, col])
                    plsc.store_scatter(rows_ref, [row_ids, col], v * kk - off)
                return carry

            lax.fori_loop(0, hpad // L, grp_body, 0)

            def row_body(r, carry):
                for i in range(D // L):
                    sl = pl.ds(i * L, L)
                    rows_ref[r, sl] = rows_ref[r, sl] * gvecs[i] + bvecs[i]
                return carry

            lax.fori_loop(0, hist, row_body, 0)

        start_gather(0, 0)
        start_gather(1, 1)

        def macro_body(i, carry):
            for b in range(NB):
                j = NB * i + b
                bp = (b + 2) % NB

                @pl.when(j >= 2)
                def _():
                    wait_out(bp)

                @pl.when(j + 2 < bpw)
                def _():
                    start_gather(bp, j + 2)

                wait_gather(b)
                compute(rows.at[b])
                start_out(b, j)
            return carry

        lax.fori_loop(0, bpw // NB, macro_body, 0)
        wait_out((bpw - 2) % NB)
        wait_out((bpw - 1) % NB)

    return k


def kernel(x, table, gamma, beta):
    b, h = x.shape
    xi = x.astype(jnp.int32)
    gb = jnp.stack([gamma, beta]).astype(jnp.float32)
    return _make_kernel(b, h)(xi, table, gb)


# speedup vs baseline: 1.2860x; 1.2860x over previous
import functools

import jax
import jax.numpy as jnp
from jax import lax
from jax.experimental import pallas as pl
from jax.experimental.pallas import tpu as pltpu
from jax.experimental.pallas import tpu_sc as plsc

D = 64
EPS = 1e-5
L = 16
NC = 2
NS = 16
NW = NC * NS
NBG = 4
NBO = 2
G1 = 128


def _rsqrt_vec(x):
    i = lax.bitcast_convert_type(x, jnp.int32)
    i = jnp.int32(0x5F375A86) - lax.shift_right_logical(i, 1)
    y = lax.bitcast_convert_type(i, jnp.float32)
    for _ in range(3):
        y = y * (jnp.float32(1.5) - jnp.float32(0.5) * x * y * y)
    return y


def _make_kernel(batch, hist):
    bpw = batch // NW
    g2 = hist - G1
    hpad = ((hist + L - 1) // L) * L
    mesh = plsc.VectorSubcoreMesh(
        core_axis_name="c", subcore_axis_name="s", num_cores=NC, num_subcores=NS
    )

    @functools.partial(
        pl.kernel,
        out_type=jax.ShapeDtypeStruct((batch, hist, D), jnp.float32),
        mesh=mesh,
        scratch_types=[
            pltpu.VMEM((bpw, hist), jnp.int32),
            pltpu.VMEM((NBG, hpad, D), jnp.float32),
            pltpu.VMEM((NBO, hpad, D), jnp.float32),
            pltpu.VMEM((2, D), jnp.float32),
            pltpu.VMEM((D, L), jnp.float32),
            pltpu.VMEM((D, L), jnp.float32),
        ]
        + [pltpu.SemaphoreType.DMA] * (NBG + NBO),
        compiler_params=pltpu.CompilerParams(
            use_tc_tiling_on_sc=False, needs_layout_passes=False
        ),
    )
    def k(x_hbm, table_hbm, gb_hbm, out_hbm, idx_all, rows, outb, gb_v,
          gdiag, bdiag, *sems):
        gsem = sems[:NBG]
        osem = sems[NBG:]
        wid = lax.axis_index("s") * NC + lax.axis_index("c")
        base = wid * bpw
        pltpu.sync_copy(x_hbm.at[pl.ds(base, bpw)], idx_all)
        pltpu.sync_copy(gb_hbm, gb_v)

        lane = lax.iota(jnp.int32, L)
        zvec = lax.bitwise_xor(lane, lane)
        for c in range(D):
            col = lax.bitwise_and(lane + c, jnp.int32(D - 1))
            gdiag[c, :] = plsc.load_gather(gb_v, [zvec, col])
            bdiag[c, :] = plsc.load_gather(gb_v, [zvec + 1, col])

        def start_gather(b, j):
            pltpu.async_copy(
                table_hbm.at[idx_all.at[j, pl.ds(0, G1)]],
                rows.at[b].at[pl.ds(0, G1)],
                gsem[b],
            )
            pltpu.async_copy(
                table_hbm.at[idx_all.at[j, pl.ds(G1, g2)]],
                rows.at[b].at[pl.ds(G1, g2)],
                gsem[b],
            )

        def wait_gather(b):
            pltpu.make_async_copy(
                table_hbm.at[pl.ds(0, hist)],
                rows.at[b].at[pl.ds(0, hist)],
                gsem[b],
            ).wait()

        def start_out(o, j):
            pltpu.async_copy(
                outb.at[o].at[pl.ds(0, hist)], out_hbm.at[base + j], osem[o]
            )

        def wait_out(o):
            pltpu.make_async_copy(
                outb.at[o].at[pl.ds(0, hist)], out_hbm.at[0], osem[o]
            ).wait()

        def compute(rows_ref, out_ref):
            def grp_body(g, carry):
                row_ids = g * L + lane
                fzero = lax.convert_element_type(zvec, jnp.float32)
                acc = [fzero] * 4
                acc2 = [fzero] * 4
                U = 4
                for cb in range(0, D, U):
                    cols = [
                        lax.bitwise_and(lane + (cb + t), jnp.int32(D - 1))
                        for t in range(U)
                    ]
                    vs = [
                        plsc.load_gather(rows_ref, [row_ids, cols[t]])
                        for t in range(U)
                    ]
                    sq = [vs[t] * vs[t] for t in range(U)]
                    acc = [acc[t] + vs[t] for t in range(U)]
                    acc2 = [acc2[t] + sq[t] for t in range(U)]
                s = (acc[0] + acc[1]) + (acc[2] + acc[3])
                q = (acc2[0] + acc2[1]) + (acc2[2] + acc2[3])
                mean = s * jnp.float32(1.0 / D)
                var = q * jnp.float32(1.0 / D) - mean * mean
                kk = _rsqrt_vec(var + jnp.float32(EPS))
                off = mean * kk
                for cb in range(0, D, U):
                    cols = [
                        lax.bitwise_and(lane + (cb + t), jnp.int32(D - 1))
                        for t in range(U)
                    ]
                    vs = [
                        plsc.load_gather(rows_ref, [row_ids, cols[t]])
                        for t in range(U)
                    ]
                    gs = [gdiag[cb + t, :] for t in range(U)]
                    bs = [bdiag[cb + t, :] for t in range(U)]
                    ys = [vs[t] * kk for t in range(U)]
                    ys = [ys[t] - off for t in range(U)]
                    ys = [ys[t] * gs[t] for t in range(U)]
                    ys = [ys[t] + bs[t] for t in range(U)]
                    for t in range(U):
                        plsc.store_scatter(out_ref, [row_ids, cols[t]], ys[t])
                return carry

            lax.fori_loop(0, hpad // L, grp_body, 0)

        start_gather(0, 0)
        start_gather(1, 1)

        def macro_body(i, carry):
            for b in range(NBG):
                j = NBG * i + b
                o = b % NBO

                @pl.when(j >= NBO)
                def _():
                    wait_out(o)

                @pl.when(j + 2 < bpw)
                def _():
                    start_gather((b + 2) % NBG, j + 2)

                wait_gather(b)
                compute(rows.at[b], outb.at[o])
                start_out(o, j)
            return carry

        lax.fori_loop(0, bpw // NBG, macro_body, 0)
        wait_out(0)
        wait_out(1)

    return k


def kernel(x, table, gamma, beta):
    b, h = x.shape
    xi = x.astype(jnp.int32)
    gb = jnp.stack([gamma, beta]).astype(jnp.float32)
    return _make_kernel(b, h)(xi, table, gb)


# speedup vs baseline: 1.4489x; 1.1267x over previous
import functools

import jax
import jax.numpy as jnp
from jax import lax
from jax.experimental import pallas as pl
from jax.experimental.pallas import tpu as pltpu
from jax.experimental.pallas import tpu_sc as plsc

D = 64
EPS = 1e-5
L = 16
NC = 2
NS = 16
NW = NC * NS
BLK = 8
G1 = 128


def _rsqrt_vec(x):
    i = lax.bitcast_convert_type(x, jnp.int32)
    i = jnp.int32(0x5F375A86) - lax.shift_right_logical(i, 1)
    y = lax.bitcast_convert_type(i, jnp.float32)
    for _ in range(3):
        y = y * (jnp.float32(1.5) - jnp.float32(0.5) * x * y * y)
    return y


def _make_kernel(batch, hist):
    bpw = batch // NW
    nblk = bpw // BLK
    g2 = hist - G1
    hpad = ((hist + L - 1) // L) * L
    nvec = list(range(0, hist - L + 1, L))
    if nvec[-1] + L < hist:
        nvec.append(hist - L)
    mesh = plsc.VectorSubcoreMesh(
        core_axis_name="c", subcore_axis_name="s", num_cores=NC, num_subcores=NS
    )

    @functools.partial(
        pl.kernel,
        out_type=jax.ShapeDtypeStruct((batch, hist, D), jnp.float32),
        mesh=mesh,
        scratch_types=[
            pltpu.VMEM((BLK, hist), jnp.int32),
            pltpu.VMEM((2, hpad), jnp.int32),
            pltpu.VMEM((2, hpad, 2 * D), jnp.float32),
            pltpu.VMEM((2, hpad, D), jnp.float32),
            pltpu.VMEM((2, D), jnp.float32),
            pltpu.VMEM((BLK, 2 * D), jnp.float32),
            pltpu.VMEM((BLK, 2 * D), jnp.float32),
        ]
        + [pltpu.SemaphoreType.DMA] * 4,
        compiler_params=pltpu.CompilerParams(
            use_tc_tiling_on_sc=True, needs_layout_passes=False
        ),
    )
    def k(x_hbm, table2_hbm, gb_hbm, out_hbm, idx8, pairb, rows, outb,
          gb_v, gdiag, bdiag, *sems):
        gsem = sems[:2]
        osem = sems[2:]
        wid = lax.axis_index("s") * NC + lax.axis_index("c")
        base = wid * bpw
        pltpu.sync_copy(gb_hbm, gb_v)

        lane = lax.iota(jnp.int32, L)
        zvec = lax.bitwise_xor(lane, lane)
        for c in range(D):
            col = lax.bitwise_and(lane + c, jnp.int32(D - 1))
            sl = pl.ds((c % BLK) * L, L)
            gdiag[c // BLK, sl] = plsc.load_gather(gb_v, [zvec, col])
            bdiag[c // BLK, sl] = plsc.load_gather(gb_v, [zvec + 1, col])

        def load_block(blk):
            pltpu.sync_copy(x_hbm.at[pl.ds(base + blk * BLK, BLK)], idx8)

        def prep_pairs(b, r):
            for o in nvec:
                iv = idx8[r, pl.ds(o, L)]
                pairb[b, pl.ds(o, L)] = lax.shift_right_logical(iv, 1)

        def start_gather(b, j):
            pltpu.async_copy(
                table2_hbm.at[pairb.at[b, pl.ds(0, G1)]],
                rows.at[b].at[pl.ds(0, G1)],
                gsem[b],
            )
            pltpu.async_copy(
                table2_hbm.at[pairb.at[b, pl.ds(G1, g2)]],
                rows.at[b].at[pl.ds(G1, g2)],
                gsem[b],
            )

        def wait_gather(b):
            pltpu.make_async_copy(
                table2_hbm.at[pl.ds(0, hist)],
                rows.at[b].at[pl.ds(0, hist)],
                gsem[b],
            ).wait()

        def start_out(o, j):
            pltpu.async_copy(
                outb.at[o].at[pl.ds(0, hist)], out_hbm.at[base + j], osem[o]
            )

        def wait_out(o):
            pltpu.make_async_copy(
                outb.at[o].at[pl.ds(0, hist)], out_hbm.at[0], osem[o]
            ).wait()

        def compute(rows_ref, out_ref, r8):
            def grp_body(g, carry):
                row_ids = g * L + lane
                idxv = idx8[r8, pl.ds(g * L, L)]
                half = lax.shift_left(
                    lax.bitwise_and(idxv, jnp.int32(1)), jnp.int32(6)
                )
                fzero = lax.convert_element_type(zvec, jnp.float32)
                acc = [fzero] * 4
                acc2 = [fzero] * 4
                U = 4
                for cb in range(0, D, U):
                    cols = [
                        half + lax.bitwise_and(lane + (cb + t), jnp.int32(D - 1))
                        for t in range(U)
                    ]
                    vs = [
                        plsc.load_gather(rows_ref, [row_ids, cols[t]])
                        for t in range(U)
                    ]
                    sq = [vs[t] * vs[t] for t in range(U)]
                    acc = [acc[t] + vs[t] for t in range(U)]
                    acc2 = [acc2[t] + sq[t] for t in range(U)]
                s = (acc[0] + acc[1]) + (acc[2] + acc[3])
                q = (acc2[0] + acc2[1]) + (acc2[2] + acc2[3])
                mean = s * jnp.float32(1.0 / D)
                var = q * jnp.float32(1.0 / D) - mean * mean
                kk = _rsqrt_vec(var + jnp.float32(EPS))
                off = mean * kk
                for cb in range(0, D, U):
                    dcols = [
                        lax.bitwise_and(lane + (cb + t), jnp.int32(D - 1))
                        for t in range(U)
                    ]
                    vs = [
                        plsc.load_gather(rows_ref, [row_ids, half + dcols[t]])
                        for t in range(U)
                    ]
                    gs = [
                        gdiag[(cb + t) // BLK, pl.ds(((cb + t) % BLK) * L, L)]
                        for t in range(U)
                    ]
                    bs = [
                        bdiag[(cb + t) // BLK, pl.ds(((cb + t) % BLK) * L, L)]
                        for t in range(U)
                    ]
                    ys = [vs[t] * kk for t in range(U)]
                    ys = [ys[t] - off for t in range(U)]
                    ys = [ys[t] * gs[t] for t in range(U)]
                    ys = [ys[t] + bs[t] for t in range(U)]
                    for t in range(U):
                        plsc.store_scatter(out_ref, [row_ids, dcols[t]], ys[t])
                return carry

            lax.fori_loop(0, hpad // L, grp_body, 0)

        load_block(0)
        prep_pairs(0, 0)
        start_gather(0, 0)

        def macro_body(ib, carry):
            for b8 in range(BLK):
                j = ib * BLK + b8
                buf = b8 % 2
                if b8 < BLK - 1:
                    prep_pairs(1 - buf, b8 + 1)
                    start_gather(1 - buf, j + 1)
                wait_gather(buf)

                @pl.when(j >= 2)
                def _():
                    wait_out(buf)

                compute(rows.at[buf], outb.at[buf], b8)
                start_out(buf, j)

            @pl.when(ib + 1 < nblk)
            def _():
                load_block(ib + 1)
                prep_pairs(0, 0)
                start_gather(0, (ib + 1) * BLK)

            return carry

        lax.fori_loop(0, nblk, macro_body, 0)
        wait_out(0)
        wait_out(1)

    return k


def kernel(x, table, gamma, beta):
    b, h = x.shape
    xi = x.astype(jnp.int32)
    table2 = table.reshape((table.shape[0] // 2, 2 * D))
    gb = jnp.stack([gamma, beta]).astype(jnp.float32)
    return _make_kernel(b, h)(xi, table2, gb)


# speedup vs baseline: 1.4546x; 1.0039x over previous
import functools

import jax
import jax.numpy as jnp
from jax import lax
from jax.experimental import pallas as pl
from jax.experimental.pallas import tpu as pltpu
from jax.experimental.pallas import tpu_sc as plsc

D = 64
EPS = 1e-5
L = 16
NC = 2
NS = 16
NW = NC * NS
BLK = 8
G1 = 128


def _rsqrt_vec(x):
    i = lax.bitcast_convert_type(x, jnp.int32)
    i = jnp.int32(0x5F375A86) - lax.shift_right_logical(i, 1)
    y = lax.bitcast_convert_type(i, jnp.float32)
    for _ in range(3):
        y = y * (jnp.float32(1.5) - jnp.float32(0.5) * x * y * y)
    return y


def _make_kernel(batch, hist):
    bpw = batch // NW
    nblk = bpw // BLK
    g2 = hist - G1
    hpad = ((hist + L - 1) // L) * L
    nvec = list(range(0, hist - L + 1, L))
    if nvec[-1] + L < hist:
        nvec.append(hist - L)
    mesh = plsc.VectorSubcoreMesh(
        core_axis_name="c", subcore_axis_name="s", num_cores=NC, num_subcores=NS
    )

    @functools.partial(
        pl.kernel,
        out_type=jax.ShapeDtypeStruct((batch, hist, D), jnp.float32),
        mesh=mesh,
        scratch_types=[
            pltpu.VMEM((BLK, hist), jnp.int32),
            pltpu.VMEM((2, hpad), jnp.int32),
            pltpu.VMEM((2, hpad, 2 * D), jnp.float32),
            pltpu.VMEM((2, hpad, D), jnp.float32),
            pltpu.VMEM((2, D), jnp.float32),
            pltpu.VMEM((BLK, 2 * D), jnp.float32),
            pltpu.VMEM((BLK, 2 * D), jnp.float32),
        ]
        + [pltpu.SemaphoreType.DMA] * 4,
        compiler_params=pltpu.CompilerParams(
            use_tc_tiling_on_sc=True, needs_layout_passes=False
        ),
    )
    def k(x_hbm, table2_hbm, gb_hbm, out_hbm, idx8, pairb, rows, outb,
          gb_v, gdiag, bdiag, *sems):
        gsem = sems[:2]
        osem = sems[2:]
        wid = lax.axis_index("s") * NC + lax.axis_index("c")
        base = wid * bpw
        pltpu.sync_copy(gb_hbm, gb_v)

        lane = lax.iota(jnp.int32, L)
        zvec = lax.bitwise_xor(lane, lane)
        for c in range(D):
            col = lax.bitwise_and(lane + c, jnp.int32(D - 1))
            sl = pl.ds((c % BLK) * L, L)
            gdiag[c // BLK, sl] = plsc.load_gather(gb_v, [zvec, col])
            bdiag[c // BLK, sl] = plsc.load_gather(gb_v, [zvec + 1, col])

        def load_block(blk):
            pltpu.sync_copy(x_hbm.at[pl.ds(base + blk * BLK, BLK)], idx8)

        def prep_pairs(b, r):
            for o in nvec:
                iv = idx8[r, pl.ds(o, L)]
                pairb[b, pl.ds(o, L)] = lax.shift_right_logical(iv, 1)

        def start_gather(b, j):
            pltpu.async_copy(
                table2_hbm.at[pairb.at[b, pl.ds(0, G1)]],
                rows.at[b].at[pl.ds(0, G1)],
                gsem[b],
            )
            pltpu.async_copy(
                table2_hbm.at[pairb.at[b, pl.ds(G1, g2)]],
                rows.at[b].at[pl.ds(G1, g2)],
                gsem[b],
            )

        def wait_gather(b):
            pltpu.make_async_copy(
                table2_hbm.at[pl.ds(0, hist)],
                rows.at[b].at[pl.ds(0, hist)],
                gsem[b],
            ).wait()

        def start_out(o, j):
            pltpu.async_copy(
                outb.at[o].at[pl.ds(0, hist)], out_hbm.at[base + j], osem[o]
            )

        def wait_out(o):
            pltpu.make_async_copy(
                outb.at[o].at[pl.ds(0, hist)], out_hbm.at[0], osem[o]
            ).wait()

        def compute(rows_ref, out_ref, r8):
            def grp_body(g, carry):
                row_ids = g * L + lane
                idxv = idx8[r8, pl.ds(g * L, L)]
                half = lax.shift_left(
                    lax.bitwise_and(idxv, jnp.int32(1)), jnp.int32(6)
                )
                fzero = lax.convert_element_type(zvec, jnp.float32)
                U = 8
                acc = [fzero] * U
                acc2 = [fzero] * U
                for cb in range(0, D, U):
                    cols = [
                        half + lax.bitwise_and(lane + (cb + t), jnp.int32(D - 1))
                        for t in range(U)
                    ]
                    vs = [
                        plsc.load_gather(rows_ref, [row_ids, cols[t]])
                        for t in range(U)
                    ]
                    sq = [vs[t] * vs[t] for t in range(U)]
                    acc = [acc[t] + vs[t] for t in range(U)]
                    acc2 = [acc2[t] + sq[t] for t in range(U)]
                while len(acc) > 1:
                    acc = [a + b for a, b in zip(acc[0::2], acc[1::2])]
                    acc2 = [a + b for a, b in zip(acc2[0::2], acc2[1::2])]
                s = acc[0]
                q = acc2[0]
                mean = s * jnp.float32(1.0 / D)
                var = q * jnp.float32(1.0 / D) - mean * mean
                kk = _rsqrt_vec(var + jnp.float32(EPS))
                off = mean * kk
                for cb in range(0, D, U):
                    dcols = [
                        lax.bitwise_and(lane + (cb + t), jnp.int32(D - 1))
                        for t in range(U)
                    ]
                    vs = [
                        plsc.load_gather(rows_ref, [row_ids, half + dcols[t]])
                        for t in range(U)
                    ]
                    gs = [
                        gdiag[(cb + t) // BLK, pl.ds(((cb + t) % BLK) * L, L)]
                        for t in range(U)
                    ]
                    bs = [
                        bdiag[(cb + t) // BLK, pl.ds(((cb + t) % BLK) * L, L)]
                        for t in range(U)
                    ]
                    ys = [vs[t] * kk for t in range(U)]
                    ys = [ys[t] - off for t in range(U)]
                    ys = [ys[t] * gs[t] for t in range(U)]
                    ys = [ys[t] + bs[t] for t in range(U)]
                    for t in range(U):
                        plsc.store_scatter(out_ref, [row_ids, dcols[t]], ys[t])
                return carry

            lax.fori_loop(0, hpad // L, grp_body, 0)

        load_block(0)
        prep_pairs(0, 0)
        start_gather(0, 0)

        def macro_body(ib, carry):
            for b8 in range(BLK):
                j = ib * BLK + b8
                buf = b8 % 2
                if b8 < BLK - 1:
                    prep_pairs(1 - buf, b8 + 1)
                    start_gather(1 - buf, j + 1)
                wait_gather(buf)

                @pl.when(j >= 2)
                def _():
                    wait_out(buf)

                compute(rows.at[buf], outb.at[buf], b8)
                start_out(buf, j)

            @pl.when(ib + 1 < nblk)
            def _():
                load_block(ib + 1)
                prep_pairs(0, 0)
                start_gather(0, (ib + 1) * BLK)

            return carry

        lax.fori_loop(0, nblk, macro_body, 0)
        wait_out(0)
        wait_out(1)

    return k


def kernel(x, table, gamma, beta):
    b, h = x.shape
    xi = x.astype(jnp.int32)
    table2 = table.reshape((table.shape[0] // 2, 2 * D))
    gb = jnp.stack([gamma, beta]).astype(jnp.float32)
    return _make_kernel(b, h)(xi, table2, gb)
